# baseline (speedup 1.0000x reference)
import jax
import jax.numpy as jnp
import numpy as np
from jax import lax
from jax.experimental import pallas as pl
from jax.experimental.pallas import tpu as pltpu

_CHO_LEN = 19
_JUNG_LEN = 21
_JONG_LEN = 27
_SPECIAL = (' ', '"', "'", '(', ')', ',', '.', '?', '0', '1', '2', '3',
            '4', '5', '6', '7', '8', '9')
_cases = [chr(10)] + list(_SPECIAL)
_TBL = np.full(len(_SPECIAL) + 2, -1, dtype=np.int32)
_TBL[:len(_cases)] = np.asarray([ord(c) for c in _cases], dtype=np.int32)
_GA = 44032

_SEGS = ((1, 20), (21, 22), (43, 28), (71, 20))
_D = 91

_Wnp = np.zeros((8, _D), dtype=np.float32)
for _s, (_off, _n) in enumerate(_SEGS):
    for _j in range(_n):
        if _j < 14:
            _Wnp[_s, _off + _j] = 2.0 ** (-_j)
        else:
            _Wnp[4 + _s, _off + _j] = 2.0 ** (-(_j - 14))


def _body(x_ref, w_ref, o_ref):
    xt = x_ref[...].T
    r = xt.shape[1]

    parts = [xt[0:1]]
    for off, n in _SEGS:
        seg = lax.slice(xt, (off, 0), (off + n, r))
        m = jnp.max(seg, axis=0, keepdims=True)
        parts.append(jnp.broadcast_to(m, (n, r)))
    mfull = jnp.concatenate(parts, axis=0)
    match = jnp.where(xt == mfull, jnp.float32(1.0), jnp.float32(0.0))

    enc = lax.dot_general(w_ref[...], match, (((1,), (0,)), ((), ())),
                          preferred_element_type=jnp.float32)
    bits = lax.bitcast_convert_type(enc, jnp.int32)
    j = jnp.int32(127) - (bits >> 23)
    lo = lax.slice(enc, (0, 0), (4, r))
    jlo = lax.slice(j, (0, 0), (4, r))
    jhi = lax.slice(j, (4, 0), (8, r)) + 14
    codes = jnp.where(lo > 0, jlo, jhi)

    cho = lax.slice(codes, (0, 0), (1, r))
    jung = lax.slice(codes, (1, 0), (2, r))
    jong = lax.slice(codes, (2, 0), (3, r))
    spec = lax.slice(codes, (3, 0), (4, r))
    han_uni = (cho * _JUNG_LEN + jung) * _JONG_LEN + jong + _GA
    spec_uni = jnp.full_like(spec, -1)
    for k in range(8, -1, -1):
        spec_uni = jnp.where(spec == k, jnp.int32(int(_TBL[k])), spec_uni)
    spec_uni = jnp.where((spec >= 9) & (spec <= 18), spec + 39, spec_uni)
    han = xt[0:1, :] >= 0.5
    o_ref[...] = jnp.where(han, han_uni, spec_uni).reshape(r)


def kernel(inputs):
    B, L, D = inputs.shape
    n_rows = B * L
    x2 = inputs.reshape(n_rows, D)
    w = jnp.asarray(_Wnp)
    r = 4096
    grid = (n_rows // r,)
    out = pl.pallas_call(
        _body,
        grid=grid,
        in_specs=[pl.BlockSpec((r, D), lambda i: (i, 0)),
                  pl.BlockSpec((8, D), lambda i: (0, 0))],
        out_specs=pl.BlockSpec((r,), lambda i: (i,)),
        out_shape=jax.ShapeDtypeStruct((n_rows,), jnp.int32),
        compiler_params=pltpu.CompilerParams(
            dimension_semantics=("parallel",)),
    )(x2, w)
    return out.reshape(B, L)


# speedup vs baseline: 1.2197x; 1.2197x over previous
import jax
import jax.numpy as jnp
import numpy as np
from jax import lax
from jax.experimental import pallas as pl
from jax.experimental.pallas import tpu as pltpu

_CHO_LEN = 19
_JUNG_LEN = 21
_JONG_LEN = 27
_SPECIAL = (' ', '"', "'", '(', ')', ',', '.', '?', '0', '1', '2', '3',
            '4', '5', '6', '7', '8', '9')
_cases = [chr(10)] + list(_SPECIAL)
_TBL = np.full(len(_SPECIAL) + 2, -1, dtype=np.int32)
_TBL[:len(_cases)] = np.asarray([ord(c) for c in _cases], dtype=np.int32)
_GA = 44032

_SEGS = ((1, 20), (21, 22), (43, 28), (71, 20))
_D = 91

_Wnp = np.zeros((8, _D), dtype=np.float32)
for _s, (_off, _n) in enumerate(_SEGS):
    for _j in range(_n):
        if _j < 14:
            _Wnp[_s, _off + _j] = 2.0 ** (-_j)
        else:
            _Wnp[4 + _s, _off + _j] = 2.0 ** (-(_j - 14))


def _body(x_ref, w_ref, o_ref):
    xt = x_ref[...].T
    r = xt.shape[1]

    parts = [xt[0:1]]
    for off, n in _SEGS:
        seg = lax.slice(xt, (off, 0), (off + n, r))
        m = jnp.max(seg, axis=0, keepdims=True)
        parts.append(jnp.broadcast_to(m, (n, r)))
    mfull = jnp.concatenate(parts, axis=0)
    match = jnp.where(xt == mfull, jnp.float32(1.0), jnp.float32(0.0))

    enc = lax.dot_general(w_ref[...], match, (((1,), (0,)), ((), ())),
                          preferred_element_type=jnp.float32)
    bits = lax.bitcast_convert_type(enc, jnp.int32)
    j = jnp.int32(127) - (bits >> 23)
    lo = lax.slice(enc, (0, 0), (4, r))
    jlo = lax.slice(j, (0, 0), (4, r))
    jhi = lax.slice(j, (4, 0), (8, r)) + 14
    codes = jnp.where(lo > 0, jlo, jhi)

    cho = lax.slice(codes, (0, 0), (1, r))
    jung = lax.slice(codes, (1, 0), (2, r))
    jong = lax.slice(codes, (2, 0), (3, r))
    spec = lax.slice(codes, (3, 0), (4, r))
    han_uni = (cho * _JUNG_LEN + jung) * _JONG_LEN + jong + _GA
    spec_uni = jnp.full_like(spec, -1)
    for k in range(8, -1, -1):
        spec_uni = jnp.where(spec == k, jnp.int32(int(_TBL[k])), spec_uni)
    spec_uni = jnp.where((spec >= 9) & (spec <= 18), spec + 39, spec_uni)
    han = xt[0:1, :] >= 0.5
    o_ref[...] = jnp.where(han, han_uni, spec_uni).reshape(r)


def kernel(inputs):
    B, L, D = inputs.shape
    n_rows = B * L
    x2 = inputs.reshape(n_rows, D)
    w = jnp.asarray(_Wnp)
    r = 16384
    grid = (n_rows // r,)
    out = pl.pallas_call(
        _body,
        grid=grid,
        in_specs=[pl.BlockSpec((r, D), lambda i: (i, 0)),
                  pl.BlockSpec((8, D), lambda i: (0, 0))],
        out_specs=pl.BlockSpec((r,), lambda i: (i,)),
        out_shape=jax.ShapeDtypeStruct((n_rows,), jnp.int32),
        compiler_params=pltpu.CompilerParams(
            dimension_semantics=("parallel",)),
    )(x2, w)
    return out.reshape(B, L)


# speedup vs baseline: 1.2556x; 1.0294x over previous
import jax
import jax.numpy as jnp
import numpy as np
from jax import lax
from jax.experimental import pallas as pl
from jax.experimental.pallas import tpu as pltpu

_CHO_LEN = 19
_JUNG_LEN = 21
_JONG_LEN = 27
_SPECIAL = (' ', '"', "'", '(', ')', ',', '.', '?', '0', '1', '2', '3',
            '4', '5', '6', '7', '8', '9')
_cases = [chr(10)] + list(_SPECIAL)
_TBL = np.full(len(_SPECIAL) + 2, -1, dtype=np.int32)
_TBL[:len(_cases)] = np.asarray([ord(c) for c in _cases], dtype=np.int32)
_GA = 44032

_SEGS = ((1, 20), (21, 22), (43, 28), (71, 20))
_D = 91

_Wnp = np.zeros((8, _D), dtype=np.float32)
for _s, (_off, _n) in enumerate(_SEGS):
    for _j in range(_n):
        if _j < 14:
            _Wnp[_s, _off + _j] = 2.0 ** (-_j)
        else:
            _Wnp[4 + _s, _off + _j] = 2.0 ** (-(_j - 14))


def _body(x_ref, w_ref, o_ref):
    xt = x_ref[...].T
    r = xt.shape[1]

    parts = [xt[0:1]]
    for off, n in _SEGS:
        seg = lax.slice(xt, (off, 0), (off + n, r))
        m = jnp.max(seg, axis=0, keepdims=True)
        parts.append(jnp.broadcast_to(m, (n, r)))
    mfull = jnp.concatenate(parts, axis=0)
    match = jnp.where(xt == mfull, jnp.float32(1.0), jnp.float32(0.0))

    enc = lax.dot_general(w_ref[...], match, (((1,), (0,)), ((), ())),
                          preferred_element_type=jnp.float32)
    bits = lax.bitcast_convert_type(enc, jnp.int32)
    j = jnp.int32(127) - (bits >> 23)
    lo = lax.slice(enc, (0, 0), (4, r))
    jlo = lax.slice(j, (0, 0), (4, r))
    jhi = lax.slice(j, (4, 0), (8, r)) + 14
    codes = jnp.where(lo > 0, jlo, jhi)

    cho = lax.slice(codes, (0, 0), (1, r))
    jung = lax.slice(codes, (1, 0), (2, r))
    jong = lax.slice(codes, (2, 0), (3, r))
    spec = lax.slice(codes, (3, 0), (4, r))
    han_uni = (cho * _JUNG_LEN + jung) * _JONG_LEN + jong + _GA
    spec_uni = jnp.full_like(spec, -1)
    for k in range(8, -1, -1):
        spec_uni = jnp.where(spec == k, jnp.int32(int(_TBL[k])), spec_uni)
    spec_uni = jnp.where((spec >= 9) & (spec <= 18), spec + 39, spec_uni)
    han = xt[0:1, :] >= 0.5
    o_ref[...] = jnp.where(han, han_uni, spec_uni).reshape(r)


def kernel(inputs):
    B, L, D = inputs.shape
    n_rows = B * L
    x2 = inputs.reshape(n_rows, D)
    w = jnp.asarray(_Wnp)
    r = 32768
    grid = (n_rows // r,)
    out = pl.pallas_call(
        _body,
        grid=grid,
        in_specs=[pl.BlockSpec((r, D), lambda i: (i, 0)),
                  pl.BlockSpec((8, D), lambda i: (0, 0))],
        out_specs=pl.BlockSpec((r,), lambda i: (i,)),
        out_shape=jax.ShapeDtypeStruct((n_rows,), jnp.int32),
        compiler_params=pltpu.CompilerParams(
            dimension_semantics=("parallel",)),
    )(x2, w)
    return out.reshape(B, L)


# speedup vs baseline: 1.2625x; 1.0056x over previous
import jax
import jax.numpy as jnp
import numpy as np
from jax import lax
from jax.experimental import pallas as pl
from jax.experimental.pallas import tpu as pltpu

_CHO_LEN = 19
_JUNG_LEN = 21
_JONG_LEN = 27
_SPECIAL = (' ', '"', "'", '(', ')', ',', '.', '?', '0', '1', '2', '3',
            '4', '5', '6', '7', '8', '9')
_cases = [chr(10)] + list(_SPECIAL)
_TBL = np.full(len(_SPECIAL) + 2, -1, dtype=np.int32)
_TBL[:len(_cases)] = np.asarray([ord(c) for c in _cases], dtype=np.int32)
_GA = 44032

_SEGS = ((1, 20), (21, 22), (43, 28), (71, 20))
_D = 91

_Wnp = np.zeros((8, _D), dtype=np.float32)
for _s, (_off, _n) in enumerate(_SEGS):
    for _j in range(_n):
        if _j < 14:
            _Wnp[_s, _off + _j] = 2.0 ** (-_j)
        else:
            _Wnp[4 + _s, _off + _j] = 2.0 ** (-(_j - 14))


def _body(x_ref, w_ref, o_ref):
    xt = x_ref[...].T
    r = xt.shape[1]

    parts = [xt[0:1]]
    for off, n in _SEGS:
        seg = lax.slice(xt, (off, 0), (off + n, r))
        m = jnp.max(seg, axis=0, keepdims=True)
        parts.append(jnp.broadcast_to(m, (n, r)))
    mfull = jnp.concatenate(parts, axis=0)
    match = jnp.where(xt == mfull, jnp.float32(1.0), jnp.float32(0.0))

    enc = lax.dot_general(w_ref[...], match, (((1,), (0,)), ((), ())),
                          preferred_element_type=jnp.float32)
    bits = lax.bitcast_convert_type(enc, jnp.int32)
    j = jnp.int32(127) - (bits >> 23)
    lo = lax.slice(enc, (0, 0), (4, r))
    jlo = lax.slice(j, (0, 0), (4, r))
    jhi = lax.slice(j, (4, 0), (8, r)) + 14
    codes = jnp.where(lo > 0, jlo, jhi)

    cho = lax.slice(codes, (0, 0), (1, r))
    jung = lax.slice(codes, (1, 0), (2, r))
    jong = lax.slice(codes, (2, 0), (3, r))
    spec = lax.slice(codes, (3, 0), (4, r))
    han_uni = (cho * _JUNG_LEN + jung) * _JONG_LEN + jong + _GA
    spec_uni = jnp.full_like(spec, -1)
    for k in range(8, -1, -1):
        spec_uni = jnp.where(spec == k, jnp.int32(int(_TBL[k])), spec_uni)
    spec_uni = jnp.where((spec >= 9) & (spec <= 18), spec + 39, spec_uni)
    han = xt[0:1, :] >= 0.5
    o_ref[...] = jnp.where(han, han_uni, spec_uni).reshape(r)


def kernel(inputs):
    B, L, D = inputs.shape
    n_rows = B * L
    x2 = inputs.reshape(n_rows, D)
    w = jnp.asarray(_Wnp)
    r = 40960
    grid = (n_rows // r,)
    out = pl.pallas_call(
        _body,
        grid=grid,
        in_specs=[pl.BlockSpec((r, D), lambda i: (i, 0)),
                  pl.BlockSpec((8, D), lambda i: (0, 0))],
        out_specs=pl.BlockSpec((r,), lambda i: (i,)),
        out_shape=jax.ShapeDtypeStruct((n_rows,), jnp.int32),
        compiler_params=pltpu.CompilerParams(
            dimension_semantics=("parallel",)),
    )(x2, w)
    return out.reshape(B, L)
